# baseline (speedup 1.0000x reference)
import jax
import jax.numpy as jnp
from jax.experimental import pallas as pl
from jax.experimental.pallas import tpu as pltpu

D = 1024
M = 64
B = 32
N = 2048
K = 2
TN = 256


def _body(idx_ref, h_ref, u_ref, writes_ref, loss_ref, w_s):
    i = pl.program_id(0)
    nblocks = pl.num_programs(0)

    @pl.when(i == 0)
    def _prep():
        def expert(e, carry):
            x = u_ref[e]
            nrm = jnp.sqrt(jnp.sum(x * x, axis=0, keepdims=True))
            y = x[:D, :] / (nrm + 1e-12)
            w_s[pl.ds(e * B, B), :] = y.T.astype(jnp.bfloat16)
            return carry

        jax.lax.fori_loop(0, M, expert, 0)

    lane_m = jax.lax.broadcasted_iota(jnp.int32, (TN, M * B), 1) // B

    h = h_ref[...]
    hsq = jnp.sum(h * h)
    hb = h.astype(jnp.bfloat16)

    zero_b = jnp.zeros((TN, M * B), dtype=jnp.bfloat16)
    a = zero_b
    cnt = jnp.zeros((TN, M * B), dtype=jnp.float32)
    for k in range(K):
        mask = lane_m == idx_ref[:, k : k + 1]
        hrep = pltpu.repeat(hb[:, k * B : (k + 1) * B], M, axis=1)
        a = a + jnp.where(mask, hrep, zero_b)
        cnt = cnt + jnp.where(mask, 1.0, 0.0)

    w = w_s[...]
    writes = jax.lax.dot_general(
        a, w, (((1,), (0,)), ((), ())), preferred_element_type=jnp.float32
    )
    writes_ref[...] = writes
    wsq = jnp.sum(writes * writes)

    r = jax.lax.dot_general(
        writes.astype(jnp.bfloat16),
        w,
        (((1,), (1,)), ((), ())),
        preferred_element_type=jnp.float32,
    )

    part = jnp.sum(cnt * r * r) + hsq - 2.0 * wsq

    @pl.when(i == 0)
    def _():
        loss_ref[0, 0] = 0.0

    loss_ref[0, 0] += part

    @pl.when(i == nblocks - 1)
    def _():
        loss_ref[0, 0] = loss_ref[0, 0] * (1.0 / (N * K * B))


@jax.jit
def kernel(h_sparse, topk_idxs, U):
    idx = topk_idxs.astype(jnp.int32)
    h2 = h_sparse.reshape(N, K * B)

    nblocks = N // TN
    writes, loss = pl.pallas_call(
        _body,
        grid=(nblocks,),
        in_specs=[
            pl.BlockSpec((TN, K), lambda i: (i, 0)),
            pl.BlockSpec((TN, K * B), lambda i: (i, 0)),
            pl.BlockSpec((M, D + 1, B), lambda i: (0, 0, 0)),
        ],
        out_specs=[
            pl.BlockSpec((TN, D), lambda i: (i, 0)),
            pl.BlockSpec((1, 1), lambda i: (0, 0), memory_space=pltpu.SMEM),
        ],
        out_shape=[
            jax.ShapeDtypeStruct((N, D), jnp.float32),
            jax.ShapeDtypeStruct((1, 1), jnp.float32),
        ],
        scratch_shapes=[pltpu.VMEM((M * B, D), jnp.bfloat16)],
    )(idx, h2, U)

    return writes, loss[0, 0]


# speedup vs baseline: 1.2113x; 1.2113x over previous
import jax
import jax.numpy as jnp
from jax.experimental import pallas as pl
from jax.experimental.pallas import tpu as pltpu

D = 1024
M = 64
B = 32
N = 2048
K = 2
TN = 512
ME = 32


def _prep_body(u_ref, w_ref):
    for e in range(ME):
        x = u_ref[e]
        nrm = jnp.sqrt(jnp.sum(x * x, axis=0, keepdims=True))
        y = x[:D, :] / (nrm + 1e-12)
        w_ref[e] = y.T.astype(jnp.bfloat16)


def _main_body(idx_ref, h_ref, w_ref, writes_ref, loss_ref):
    i = pl.program_id(0)
    nblocks = pl.num_programs(0)

    lane_m = jax.lax.broadcasted_iota(jnp.int32, (TN, M * B), 1) // B

    h = h_ref[...]
    hsq = jnp.sum(h * h)
    hb = h.astype(jnp.bfloat16)

    zero_b = jnp.zeros((TN, M * B), dtype=jnp.bfloat16)
    a = zero_b
    cnt = jnp.zeros((TN, M * B), dtype=jnp.float32)
    for k in range(K):
        mask = lane_m == idx_ref[:, k : k + 1]
        hrep = pltpu.repeat(hb[:, k * B : (k + 1) * B], M, axis=1)
        a = a + jnp.where(mask, hrep, zero_b)
        cnt = cnt + jnp.where(mask, 1.0, 0.0)

    w = w_ref[...]
    writes = jax.lax.dot_general(
        a, w, (((1,), (0,)), ((), ())), preferred_element_type=jnp.float32
    )
    writes_ref[...] = writes
    wsq = jnp.sum(writes * writes)

    r = jax.lax.dot_general(
        writes.astype(jnp.bfloat16),
        w,
        (((1,), (1,)), ((), ())),
        preferred_element_type=jnp.float32,
    )

    part = jnp.sum(cnt * r * r) + hsq - 2.0 * wsq

    @pl.when(i == 0)
    def _():
        loss_ref[0, 0] = 0.0

    loss_ref[0, 0] += part

    @pl.when(i == nblocks - 1)
    def _():
        loss_ref[0, 0] = loss_ref[0, 0] * (1.0 / (N * K * B))


@jax.jit
def kernel(h_sparse, topk_idxs, U):
    idx = topk_idxs.astype(jnp.int32)
    h2 = h_sparse.reshape(N, K * B)

    wt = pl.pallas_call(
        _prep_body,
        grid=(M // ME,),
        in_specs=[pl.BlockSpec((ME, D + 1, B), lambda m: (m, 0, 0))],
        out_specs=pl.BlockSpec((ME, B, D), lambda m: (m, 0, 0)),
        out_shape=jax.ShapeDtypeStruct((M, B, D), jnp.bfloat16),
    )(U)
    w2 = wt.reshape(M * B, D)

    nblocks = N // TN
    writes, loss = pl.pallas_call(
        _main_body,
        grid=(nblocks,),
        in_specs=[
            pl.BlockSpec((TN, K), lambda i: (i, 0)),
            pl.BlockSpec((TN, K * B), lambda i: (i, 0)),
            pl.BlockSpec((M * B, D), lambda i: (0, 0)),
        ],
        out_specs=[
            pl.BlockSpec((TN, D), lambda i: (i, 0)),
            pl.BlockSpec((1, 1), lambda i: (0, 0), memory_space=pltpu.SMEM),
        ],
        out_shape=[
            jax.ShapeDtypeStruct((N, D), jnp.float32),
            jax.ShapeDtypeStruct((1, 1), jnp.float32),
        ],
    )(idx, h2, w2)

    return writes, loss[0, 0]
